# Initial kernel scaffold; baseline (speedup 1.0000x reference)
import jax
import jax.numpy as jnp
from jax.experimental import pallas as pl


def kernel(x, W_enc, b_enc, W_dec, b_dec):
    raise NotImplementedError("write your pallas kernel here")


# speedup vs baseline: 20.4105x; 20.4105x over previous
import functools

import jax
import jax.numpy as jnp
from jax import lax
from jax.experimental import pallas as pl
from jax.experimental.pallas import tpu as pltpu
from jax.experimental.pallas import tpu_sc as plsc

_ACT_DIM = 2048
_DICT = 16384
_BATCH = 2048
_KSEL = 64 * _BATCH
_NTOT = _BATCH * _DICT


_BM_E = 1024
_BN_E = 1024


def _enc_body(x_ref, w_ref, benc_ref, bdec_ref, out_ref):
    xb = x_ref[...] - bdec_ref[...]
    acts = jnp.dot(xb, w_ref[...], preferred_element_type=jnp.float32)
    out_ref[...] = jnp.maximum(acts + benc_ref[...], 0.0)


_encode = pl.pallas_call(
    _enc_body,
    grid=(_DICT // _BN_E, _BATCH // _BM_E),
    in_specs=[
        pl.BlockSpec((_BM_E, _ACT_DIM), lambda n, m: (m, 0)),
        pl.BlockSpec((_ACT_DIM, _BN_E), lambda n, m: (0, n)),
        pl.BlockSpec((1, _BN_E), lambda n, m: (0, n)),
        pl.BlockSpec((1, _ACT_DIM), lambda n, m: (0, 0)),
    ],
    out_specs=pl.BlockSpec((_BM_E, _BN_E), lambda n, m: (m, n)),
    out_shape=jax.ShapeDtypeStruct((_BATCH, _DICT), jnp.float32),
)


_NB1 = 32768
_NB2 = 65536
_NTILES = 32
_PER_TILE = _NTOT // _NTILES
_CHUNK = 16384
_NCHUNK = _PER_TILE // _CHUNK

@functools.lru_cache(maxsize=None)
def _sc_mesh():
    return plsc.VectorSubcoreMesh(core_axis_name="c", subcore_axis_name="s")


def _zero_vmem(ref, nwords):
    z = jnp.zeros((16,), jnp.float32)

    def body(i, c):
        ref[pl.ds(i * 16, 16)] = z
        return c

    lax.fori_loop(0, nwords // 16, body, 0)


def _hist1_body(acts_hbm, out_hbm, buf, hist):
    wid = lax.axis_index("s") * 2 + lax.axis_index("c")
    _zero_vmem(hist, _NB1)
    base = wid * _PER_TILE
    ones = jnp.ones((16,), jnp.float32)
    sh16 = jnp.full((16,), 16, jnp.int32)

    def chunk(ci, c):
        pltpu.sync_copy(acts_hbm.at[pl.ds(base + ci * _CHUNK, _CHUNK)], buf)

        def vbody(i, cc):
            for j in range(4):
                v = buf[pl.ds(i * 64 + j * 16, 16)]
                bits = plsc.bitcast(v, jnp.int32)
                bucket = lax.shift_right_logical(bits, sh16)
                m = bits != 0
                plsc.addupdate_scatter(hist, [bucket], ones, mask=m)
            return cc

        lax.fori_loop(0, _CHUNK // 64, vbody, 0)
        return c

    lax.fori_loop(0, _NCHUNK, chunk, 0)
    pltpu.sync_copy(hist, out_hbm.at[wid])


@functools.lru_cache(maxsize=None)
def _hist1():
    return pl.kernel(
        _hist1_body,
        out_type=jax.ShapeDtypeStruct((_NTILES, _NB1), jnp.float32),
        mesh=_sc_mesh(),
        compiler_params=pltpu.CompilerParams(needs_layout_passes=False),
        scratch_types=[
            pltpu.VMEM((_CHUNK,), jnp.float32),
            pltpu.VMEM((_NB1,), jnp.float32),
        ],
    )


def _hist2_body(acts_hbm, h_hbm, out_hbm, buf, hist, hv):
    wid = lax.axis_index("s") * 2 + lax.axis_index("c")
    _zero_vmem(hist, _NB2)
    pltpu.sync_copy(h_hbm, hv)
    hvec = hv[...]
    base = wid * _PER_TILE
    ones = jnp.ones((16,), jnp.float32)
    sh16 = jnp.full((16,), 16, jnp.int32)
    lowmask = jnp.full((16,), 0xFFFF, jnp.int32)

    def chunk(ci, c):
        pltpu.sync_copy(acts_hbm.at[pl.ds(base + ci * _CHUNK, _CHUNK)], buf)

        def vbody(i, cc):
            for j in range(4):
                v = buf[pl.ds(i * 64 + j * 16, 16)]
                bits = plsc.bitcast(v, jnp.int32)
                hi = lax.shift_right_logical(bits, sh16)
                m = jnp.logical_and(hi == hvec, bits != 0)
                low = jnp.bitwise_and(bits, lowmask)
                plsc.addupdate_scatter(hist, [low], ones, mask=m)
            return cc

        lax.fori_loop(0, _CHUNK // 64, vbody, 0)
        return c

    lax.fori_loop(0, _NCHUNK, chunk, 0)
    pltpu.sync_copy(hist, out_hbm.at[wid])


@functools.lru_cache(maxsize=None)
def _hist2():
    return pl.kernel(
        _hist2_body,
        out_type=jax.ShapeDtypeStruct((_NTILES, _NB2), jnp.float32),
        mesh=_sc_mesh(),
        compiler_params=pltpu.CompilerParams(needs_layout_passes=False),
        scratch_types=[
            pltpu.VMEM((_CHUNK,), jnp.float32),
            pltpu.VMEM((_NB2,), jnp.float32),
            pltpu.VMEM((16,), jnp.int32),
        ],
    )


def _above_counts(Hs, nrows):
    rs = jnp.sum(Hs, axis=1, keepdims=True)
    ii = lax.broadcasted_iota(jnp.int32, (nrows, nrows), 0)
    jj = lax.broadcasted_iota(jnp.int32, (nrows, nrows), 1)
    U = (jj > ii).astype(jnp.float32)
    r_after = jnp.dot(U, rs, preferred_element_type=jnp.float32,
                      precision=lax.Precision.HIGHEST)
    aa = lax.broadcasted_iota(jnp.int32, (128, 128), 0)
    bb = lax.broadcasted_iota(jnp.int32, (128, 128), 1)
    V = (aa > bb).astype(jnp.float32)
    w_after = jnp.dot(Hs, V, preferred_element_type=jnp.float32,
                      precision=lax.Precision.HIGHEST)
    return r_after + w_after


def _flat_iota(nrows):
    fi = (lax.broadcasted_iota(jnp.int32, (nrows, 128), 0) * 128
          + lax.broadcasted_iota(jnp.int32, (nrows, 128), 1))
    return fi.astype(jnp.float32)


def _sel1_body(hall_ref, out_ref):
    Hs = jnp.sum(hall_ref[...], axis=0)
    C = _above_counts(Hs, 256)
    kf = float(_KSEL)
    cond = jnp.logical_and(C < kf, C + Hs >= kf)
    h_f = jnp.sum(jnp.where(cond, _flat_iota(256), 0.0))
    a_f = jnp.sum(jnp.where(cond, C, 0.0))
    rio = lax.broadcasted_iota(jnp.int32, (8, 128), 0)
    out_ref[...] = jnp.where(
        rio == 0, h_f.astype(jnp.int32),
        jnp.where(rio == 1, a_f.astype(jnp.int32), 0))


_sel1 = pl.pallas_call(
    _sel1_body,
    out_shape=jax.ShapeDtypeStruct((8, 128), jnp.int32),
)


def _sel2_body(h2_ref, info_ref, out_ref):
    Hs = jnp.sum(h2_ref[...], axis=0)
    C = _above_counts(Hs, 512)
    h = info_ref[0, 0]
    a = info_ref[1, 0]
    kk = float(_KSEL) - a.astype(jnp.float32)
    cond = jnp.logical_and(C < kk, C + Hs >= kk)
    l_f = jnp.sum(jnp.where(cond, _flat_iota(512), 0.0))
    tau_bits = h * 65536 + l_f.astype(jnp.int32)
    out_ref[...] = lax.bitcast_convert_type(
        jnp.full((8, 128), tau_bits, jnp.int32), jnp.float32)


_sel2 = pl.pallas_call(
    _sel2_body,
    out_shape=jax.ShapeDtypeStruct((8, 128), jnp.float32),
)


_BM_D = 1024
_BK_D = 512


def _dec_body(tau_ref, acts_ref, w_ref, bdec_ref, out_ref):
    kb = pl.program_id(1)
    tau = tau_ref[0, 0]
    a = acts_ref[...]
    a = jnp.where(a >= tau, a, 0.0)
    part = jnp.dot(a, w_ref[...], preferred_element_type=jnp.float32,
                   precision=lax.Precision.HIGHEST)

    @pl.when(kb == 0)
    def _():
        out_ref[...] = part + bdec_ref[...]

    @pl.when(kb > 0)
    def _():
        out_ref[...] += part


_decode = pl.pallas_call(
    _dec_body,
    grid=(_BATCH // _BM_D, _DICT // _BK_D),
    in_specs=[
        pl.BlockSpec((8, 128), lambda m, k: (0, 0)),
        pl.BlockSpec((_BM_D, _BK_D), lambda m, k: (m, k)),
        pl.BlockSpec((_BK_D, _ACT_DIM), lambda m, k: (k, 0)),
        pl.BlockSpec((1, _ACT_DIM), lambda m, k: (0, 0)),
    ],
    out_specs=pl.BlockSpec((_BM_D, _ACT_DIM), lambda m, k: (m, 0)),
    out_shape=jax.ShapeDtypeStruct((_BATCH, _ACT_DIM), jnp.float32),
)


def kernel(x, W_enc, b_enc, W_dec, b_dec):
    acts = _encode(x, W_enc, b_enc.reshape(1, -1), b_dec.reshape(1, -1))
    acts_flat = acts.reshape(-1)
    h1 = _hist1()(acts_flat)
    info = _sel1(h1.reshape(_NTILES, 256, 128))
    h_arr = info[0, :16]
    h2 = _hist2()(acts_flat, h_arr)
    tau = _sel2(h2.reshape(_NTILES, 512, 128), info)
    return _decode(tau, acts, W_dec, b_dec.reshape(1, -1))


# speedup vs baseline: 22.2230x; 1.0888x over previous
import functools

import jax
import jax.numpy as jnp
from jax import lax
from jax.experimental import pallas as pl
from jax.experimental.pallas import tpu as pltpu
from jax.experimental.pallas import tpu_sc as plsc

_ACT_DIM = 2048
_DICT = 16384
_BATCH = 2048
_KSEL = 64 * _BATCH
_NTOT = _BATCH * _DICT


_BM_E = 1024
_BN_E = 1024


def _enc_body(x_ref, w_ref, benc_ref, bdec_ref, out_ref):
    xb = x_ref[...] - bdec_ref[...]
    acts = jnp.dot(xb, w_ref[...], preferred_element_type=jnp.float32)
    out_ref[...] = jnp.maximum(acts + benc_ref[...], 0.0)


_encode = pl.pallas_call(
    _enc_body,
    grid=(_DICT // _BN_E, _BATCH // _BM_E),
    in_specs=[
        pl.BlockSpec((_BM_E, _ACT_DIM), lambda n, m: (m, 0)),
        pl.BlockSpec((_ACT_DIM, _BN_E), lambda n, m: (0, n)),
        pl.BlockSpec((1, _BN_E), lambda n, m: (0, n)),
        pl.BlockSpec((1, _ACT_DIM), lambda n, m: (0, 0)),
    ],
    out_specs=pl.BlockSpec((_BM_E, _BN_E), lambda n, m: (m, n)),
    out_shape=jax.ShapeDtypeStruct((_BATCH, _DICT), jnp.float32),
)


_NB1 = 32768
_NB2 = 65536
_NTILES = 32
_PER_TILE = _NTOT // _NTILES
_CHUNK = 16384
_NCHUNK = _PER_TILE // _CHUNK

@functools.lru_cache(maxsize=None)
def _sc_mesh():
    return plsc.VectorSubcoreMesh(core_axis_name="c", subcore_axis_name="s")


def _make_hist_body(pass2):
    nbins = _NB2 if pass2 else _NB1

    def body(*args):
        if pass2:
            acts_hbm, h_hbm, out_hbm, buf0, buf1, hist, hv, sem0, sem1 = args
        else:
            acts_hbm, out_hbm, buf0, buf1, hist, sem0, sem1 = args
        wid = lax.axis_index("s") * 2 + lax.axis_index("c")

        z = jnp.zeros((16,), jnp.float32)

        def zb(i, c):
            for j in range(8):
                hist[pl.ds(i * 128 + j * 16, 16)] = z
            return c

        lax.fori_loop(0, nbins // 128, zb, 0)

        if pass2:
            pltpu.sync_copy(h_hbm, hv)
            hvec = hv[...]
        base = wid * _PER_TILE
        ones = jnp.ones((16,), jnp.float32)
        sh16 = jnp.full((16,), 16, jnp.int32)
        lowm = jnp.full((16,), 0xFFFF, jnp.int32)

        def process(buf):
            def vb(i, c):
                for j in range(8):
                    v = buf[pl.ds(i * 128 + j * 16, 16)]
                    bits = plsc.bitcast(v, jnp.int32)
                    if pass2:
                        hi = lax.shift_right_logical(bits, sh16)
                        m = jnp.logical_and(hi == hvec, bits != 0)
                        idx = jnp.bitwise_and(bits, lowm)
                    else:
                        idx = lax.shift_right_logical(bits, sh16)
                        m = bits != 0
                    plsc.addupdate_scatter(hist, [idx], ones, mask=m)
                return c

            lax.fori_loop(0, _CHUNK // 128, vb, 0)

        def start(ci, buf, sem):
            pltpu.async_copy(
                acts_hbm.at[pl.ds(base + ci * _CHUNK, _CHUNK)], buf, sem)

        def wait(buf, sem):
            pltpu.make_async_copy(
                acts_hbm.at[pl.ds(base, _CHUNK)], buf, sem).wait()

        start(0, buf0, sem0)

        def pair(i, c):
            a = 2 * i
            wait(buf0, sem0)
            start(a + 1, buf1, sem1)
            process(buf0)
            wait(buf1, sem1)

            @pl.when(a + 2 < _NCHUNK)
            def _():
                start(a + 2, buf0, sem0)

            process(buf1)
            return c

        lax.fori_loop(0, _NCHUNK // 2, pair, 0)
        pltpu.sync_copy(hist, out_hbm.at[wid])

    return body


_hist1_body = _make_hist_body(False)
_hist2_body = _make_hist_body(True)


@functools.lru_cache(maxsize=None)
def _hist1():
    return pl.kernel(
        _hist1_body,
        out_type=jax.ShapeDtypeStruct((_NTILES, _NB1), jnp.float32),
        mesh=_sc_mesh(),
        compiler_params=pltpu.CompilerParams(needs_layout_passes=False),
        scratch_types=[
            pltpu.VMEM((_CHUNK,), jnp.float32),
            pltpu.VMEM((_CHUNK,), jnp.float32),
            pltpu.VMEM((_NB1,), jnp.float32),
            pltpu.SemaphoreType.DMA,
            pltpu.SemaphoreType.DMA,
        ],
    )


@functools.lru_cache(maxsize=None)
def _hist2():
    return pl.kernel(
        _hist2_body,
        out_type=jax.ShapeDtypeStruct((_NTILES, _NB2), jnp.float32),
        mesh=_sc_mesh(),
        compiler_params=pltpu.CompilerParams(needs_layout_passes=False),
        scratch_types=[
            pltpu.VMEM((_CHUNK,), jnp.float32),
            pltpu.VMEM((_CHUNK,), jnp.float32),
            pltpu.VMEM((_NB2,), jnp.float32),
            pltpu.VMEM((16,), jnp.int32),
            pltpu.SemaphoreType.DMA,
            pltpu.SemaphoreType.DMA,
        ],
    )


def _above_counts(Hs, nrows):
    rs = jnp.sum(Hs, axis=1, keepdims=True)
    ii = lax.broadcasted_iota(jnp.int32, (nrows, nrows), 0)
    jj = lax.broadcasted_iota(jnp.int32, (nrows, nrows), 1)
    U = (jj > ii).astype(jnp.float32)
    r_after = jnp.dot(U, rs, preferred_element_type=jnp.float32,
                      precision=lax.Precision.HIGHEST)
    aa = lax.broadcasted_iota(jnp.int32, (128, 128), 0)
    bb = lax.broadcasted_iota(jnp.int32, (128, 128), 1)
    V = (aa > bb).astype(jnp.float32)
    w_after = jnp.dot(Hs, V, preferred_element_type=jnp.float32,
                      precision=lax.Precision.HIGHEST)
    return r_after + w_after


def _flat_iota(nrows):
    fi = (lax.broadcasted_iota(jnp.int32, (nrows, 128), 0) * 128
          + lax.broadcasted_iota(jnp.int32, (nrows, 128), 1))
    return fi.astype(jnp.float32)


def _sel1_body(hall_ref, out_ref):
    Hs = jnp.sum(hall_ref[...], axis=0)
    C = _above_counts(Hs, 256)
    kf = float(_KSEL)
    cond = jnp.logical_and(C < kf, C + Hs >= kf)
    h_f = jnp.sum(jnp.where(cond, _flat_iota(256), 0.0))
    a_f = jnp.sum(jnp.where(cond, C, 0.0))
    rio = lax.broadcasted_iota(jnp.int32, (8, 128), 0)
    out_ref[...] = jnp.where(
        rio == 0, h_f.astype(jnp.int32),
        jnp.where(rio == 1, a_f.astype(jnp.int32), 0))


_sel1 = pl.pallas_call(
    _sel1_body,
    out_shape=jax.ShapeDtypeStruct((8, 128), jnp.int32),
)


def _sel2_body(h2_ref, info_ref, out_ref):
    Hs = jnp.sum(h2_ref[...], axis=0)
    C = _above_counts(Hs, 512)
    h = info_ref[0, 0]
    a = info_ref[1, 0]
    kk = float(_KSEL) - a.astype(jnp.float32)
    cond = jnp.logical_and(C < kk, C + Hs >= kk)
    l_f = jnp.sum(jnp.where(cond, _flat_iota(512), 0.0))
    tau_bits = h * 65536 + l_f.astype(jnp.int32)
    out_ref[...] = lax.bitcast_convert_type(
        jnp.full((8, 128), tau_bits, jnp.int32), jnp.float32)


_sel2 = pl.pallas_call(
    _sel2_body,
    out_shape=jax.ShapeDtypeStruct((8, 128), jnp.float32),
)


_BM_D = 1024
_BK_D = 512


def _dec_body(tau_ref, acts_ref, w_ref, bdec_ref, out_ref):
    kb = pl.program_id(1)
    tau = tau_ref[0, 0]
    a = acts_ref[...]
    a = jnp.where(a >= tau, a, 0.0)
    part = jnp.dot(a, w_ref[...], preferred_element_type=jnp.float32,
                   precision=lax.Precision.HIGHEST)

    @pl.when(kb == 0)
    def _():
        out_ref[...] = part + bdec_ref[...]

    @pl.when(kb > 0)
    def _():
        out_ref[...] += part


_decode = pl.pallas_call(
    _dec_body,
    grid=(_BATCH // _BM_D, _DICT // _BK_D),
    in_specs=[
        pl.BlockSpec((8, 128), lambda m, k: (0, 0)),
        pl.BlockSpec((_BM_D, _BK_D), lambda m, k: (m, k)),
        pl.BlockSpec((_BK_D, _ACT_DIM), lambda m, k: (k, 0)),
        pl.BlockSpec((1, _ACT_DIM), lambda m, k: (0, 0)),
    ],
    out_specs=pl.BlockSpec((_BM_D, _ACT_DIM), lambda m, k: (m, 0)),
    out_shape=jax.ShapeDtypeStruct((_BATCH, _ACT_DIM), jnp.float32),
)


def kernel(x, W_enc, b_enc, W_dec, b_dec):
    acts = _encode(x, W_enc, b_enc.reshape(1, -1), b_dec.reshape(1, -1))
    acts_flat = acts.reshape(-1)
    h1 = _hist1()(acts_flat)
    info = _sel1(h1.reshape(_NTILES, 256, 128))
    h_arr = info[0, :16]
    h2 = _hist2()(acts_flat, h_arr)
    tau = _sel2(h2.reshape(_NTILES, 512, 128), info)
    return _decode(tau, acts, W_dec, b_dec.reshape(1, -1))


# speedup vs baseline: 60.0078x; 2.7003x over previous
import functools

import jax
import jax.numpy as jnp
from jax import lax
from jax.experimental import pallas as pl
from jax.experimental.pallas import tpu as pltpu
from jax.experimental.pallas import tpu_sc as plsc

_ACT_DIM = 2048
_DICT = 16384
_BATCH = 2048
_KSEL = 64 * _BATCH
_NTOT = _BATCH * _DICT


_BM_E = 1024
_BN_E = 1024


def _enc_body(x_ref, w_ref, benc_ref, bdec_ref, out_ref):
    xb = x_ref[...] - bdec_ref[...]
    acts = jnp.dot(xb, w_ref[...], preferred_element_type=jnp.float32)
    out_ref[...] = jnp.maximum(acts + benc_ref[...], 0.0)


_encode = pl.pallas_call(
    _enc_body,
    grid=(_DICT // _BN_E, _BATCH // _BM_E),
    in_specs=[
        pl.BlockSpec((_BM_E, _ACT_DIM), lambda n, m: (m, 0)),
        pl.BlockSpec((_ACT_DIM, _BN_E), lambda n, m: (0, n)),
        pl.BlockSpec((1, _BN_E), lambda n, m: (0, n)),
        pl.BlockSpec((1, _ACT_DIM), lambda n, m: (0, 0)),
    ],
    out_specs=pl.BlockSpec((_BM_E, _BN_E), lambda n, m: (m, n)),
    out_shape=jax.ShapeDtypeStruct((_BATCH, _DICT), jnp.float32),
)


_NB1 = 32768
_NB2 = 65536
_NTILES = 32
_PER_TILE = _NTOT // _NTILES
_CHUNK = 16384
_NCHUNK = _PER_TILE // _CHUNK

@functools.lru_cache(maxsize=None)
def _sc_mesh():
    return plsc.VectorSubcoreMesh(core_axis_name="c", subcore_axis_name="s")


def _make_hist_body(pass2):
    nbins = _NB2 if pass2 else _NB1

    def body(*args):
        if pass2:
            acts_hbm, h_hbm, out_hbm, buf0, buf1, hist, hv, sem0, sem1 = args
        else:
            acts_hbm, out_hbm, buf0, buf1, hist, sem0, sem1 = args
        wid = lax.axis_index("s") * 2 + lax.axis_index("c")

        z = jnp.zeros((16,), jnp.float32)

        @plsc.parallel_loop(0, nbins // 16, unroll=8)
        def _(i):
            hist[pl.ds(i * 16, 16)] = z

        if pass2:
            pltpu.sync_copy(h_hbm, hv)
            hvec = hv[...]
        base = wid * _PER_TILE
        ones = jnp.ones((16,), jnp.float32)
        sh16 = jnp.full((16,), 16, jnp.int32)
        lowm = jnp.full((16,), 0xFFFF, jnp.int32)

        def process(buf):
            @plsc.parallel_loop(0, _CHUNK // 16, unroll=8)
            def _(i):
                v = buf[pl.ds(i * 16, 16)]
                bits = plsc.bitcast(v, jnp.int32)
                if pass2:
                    hi = lax.shift_right_logical(bits, sh16)
                    m = jnp.logical_and(hi == hvec, bits != 0)
                    idx = jnp.bitwise_and(bits, lowm)
                else:
                    idx = lax.shift_right_logical(bits, sh16)
                    m = bits != 0
                plsc.addupdate_scatter(hist, [idx], ones, mask=m)

        def start(ci, buf, sem):
            pltpu.async_copy(
                acts_hbm.at[pl.ds(base + ci * _CHUNK, _CHUNK)], buf, sem)

        def wait(buf, sem):
            pltpu.make_async_copy(
                acts_hbm.at[pl.ds(base, _CHUNK)], buf, sem).wait()

        start(0, buf0, sem0)

        def pair(i, c):
            a = 2 * i
            wait(buf0, sem0)
            start(a + 1, buf1, sem1)
            process(buf0)
            wait(buf1, sem1)

            @pl.when(a + 2 < _NCHUNK)
            def _():
                start(a + 2, buf0, sem0)

            process(buf1)
            return c

        lax.fori_loop(0, _NCHUNK // 2, pair, 0)
        pltpu.sync_copy(hist, out_hbm.at[wid])

    return body


_hist1_body = _make_hist_body(False)
_hist2_body = _make_hist_body(True)


@functools.lru_cache(maxsize=None)
def _hist1():
    return pl.kernel(
        _hist1_body,
        out_type=jax.ShapeDtypeStruct((_NTILES, _NB1), jnp.float32),
        mesh=_sc_mesh(),
        compiler_params=pltpu.CompilerParams(needs_layout_passes=False),
        scratch_types=[
            pltpu.VMEM((_CHUNK,), jnp.float32),
            pltpu.VMEM((_CHUNK,), jnp.float32),
            pltpu.VMEM((_NB1,), jnp.float32),
            pltpu.SemaphoreType.DMA,
            pltpu.SemaphoreType.DMA,
        ],
    )


@functools.lru_cache(maxsize=None)
def _hist2():
    return pl.kernel(
        _hist2_body,
        out_type=jax.ShapeDtypeStruct((_NTILES, _NB2), jnp.float32),
        mesh=_sc_mesh(),
        compiler_params=pltpu.CompilerParams(needs_layout_passes=False),
        scratch_types=[
            pltpu.VMEM((_CHUNK,), jnp.float32),
            pltpu.VMEM((_CHUNK,), jnp.float32),
            pltpu.VMEM((_NB2,), jnp.float32),
            pltpu.VMEM((16,), jnp.int32),
            pltpu.SemaphoreType.DMA,
            pltpu.SemaphoreType.DMA,
        ],
    )


def _above_counts(Hs, nrows):
    rs = jnp.sum(Hs, axis=1, keepdims=True)
    ii = lax.broadcasted_iota(jnp.int32, (nrows, nrows), 0)
    jj = lax.broadcasted_iota(jnp.int32, (nrows, nrows), 1)
    U = (jj > ii).astype(jnp.float32)
    r_after = jnp.dot(U, rs, preferred_element_type=jnp.float32,
                      precision=lax.Precision.HIGHEST)
    aa = lax.broadcasted_iota(jnp.int32, (128, 128), 0)
    bb = lax.broadcasted_iota(jnp.int32, (128, 128), 1)
    V = (aa > bb).astype(jnp.float32)
    w_after = jnp.dot(Hs, V, preferred_element_type=jnp.float32,
                      precision=lax.Precision.HIGHEST)
    return r_after + w_after


def _flat_iota(nrows):
    fi = (lax.broadcasted_iota(jnp.int32, (nrows, 128), 0) * 128
          + lax.broadcasted_iota(jnp.int32, (nrows, 128), 1))
    return fi.astype(jnp.float32)


def _sel1_body(hall_ref, out_ref):
    Hs = jnp.sum(hall_ref[...], axis=0)
    C = _above_counts(Hs, 256)
    kf = float(_KSEL)
    cond = jnp.logical_and(C < kf, C + Hs >= kf)
    h_f = jnp.sum(jnp.where(cond, _flat_iota(256), 0.0))
    a_f = jnp.sum(jnp.where(cond, C, 0.0))
    rio = lax.broadcasted_iota(jnp.int32, (8, 128), 0)
    out_ref[...] = jnp.where(
        rio == 0, h_f.astype(jnp.int32),
        jnp.where(rio == 1, a_f.astype(jnp.int32), 0))


_sel1 = pl.pallas_call(
    _sel1_body,
    out_shape=jax.ShapeDtypeStruct((8, 128), jnp.int32),
)


def _sel2_body(h2_ref, info_ref, out_ref):
    Hs = jnp.sum(h2_ref[...], axis=0)
    C = _above_counts(Hs, 512)
    h = info_ref[0, 0]
    a = info_ref[1, 0]
    kk = float(_KSEL) - a.astype(jnp.float32)
    cond = jnp.logical_and(C < kk, C + Hs >= kk)
    l_f = jnp.sum(jnp.where(cond, _flat_iota(512), 0.0))
    tau_bits = h * 65536 + l_f.astype(jnp.int32)
    out_ref[...] = lax.bitcast_convert_type(
        jnp.full((8, 128), tau_bits, jnp.int32), jnp.float32)


_sel2 = pl.pallas_call(
    _sel2_body,
    out_shape=jax.ShapeDtypeStruct((8, 128), jnp.float32),
)


_BM_D = 1024
_BK_D = 1024


def _dec_body(tau_ref, acts_ref, w_ref, bdec_ref, out_ref):
    kb = pl.program_id(1)
    tau = tau_ref[0, 0]
    a = acts_ref[...]
    a = jnp.where(a >= tau, a, 0.0).astype(jnp.bfloat16)
    part = jnp.dot(a, w_ref[...], preferred_element_type=jnp.float32)

    @pl.when(kb == 0)
    def _():
        out_ref[...] = part + bdec_ref[...]

    @pl.when(kb > 0)
    def _():
        out_ref[...] += part


_decode = pl.pallas_call(
    _dec_body,
    grid=(_BATCH // _BM_D, _DICT // _BK_D),
    in_specs=[
        pl.BlockSpec((8, 128), lambda m, k: (0, 0)),
        pl.BlockSpec((_BM_D, _BK_D), lambda m, k: (m, k)),
        pl.BlockSpec((_BK_D, _ACT_DIM), lambda m, k: (k, 0)),
        pl.BlockSpec((1, _ACT_DIM), lambda m, k: (0, 0)),
    ],
    out_specs=pl.BlockSpec((_BM_D, _ACT_DIM), lambda m, k: (m, 0)),
    out_shape=jax.ShapeDtypeStruct((_BATCH, _ACT_DIM), jnp.float32),
)


def kernel(x, W_enc, b_enc, W_dec, b_dec):
    acts = _encode(x, W_enc, b_enc.reshape(1, -1), b_dec.reshape(1, -1))
    acts_flat = acts.reshape(-1)
    h1 = _hist1()(acts_flat)
    info = _sel1(h1.reshape(_NTILES, 256, 128))
    h_arr = info[0, :16]
    h2 = _hist2()(acts_flat, h_arr)
    tau = _sel2(h2.reshape(_NTILES, 512, 128), info)
    return _decode(tau, acts, W_dec.astype(jnp.bfloat16), b_dec.reshape(1, -1))


# speedup vs baseline: 73.2918x; 1.2214x over previous
import functools

import jax
import jax.numpy as jnp
from jax import lax
from jax.experimental import pallas as pl
from jax.experimental.pallas import tpu as pltpu
from jax.experimental.pallas import tpu_sc as plsc

_ACT_DIM = 2048
_DICT = 16384
_BATCH = 2048
_KSEL = 64 * _BATCH
_NTOT = _BATCH * _DICT


_BM_E = 1024
_BN_E = 1024


def _enc_body(x_ref, w_ref, benc_ref, bdec_ref, out_ref):
    xb = x_ref[...] - bdec_ref[...]
    acts = jnp.dot(xb, w_ref[...], preferred_element_type=jnp.float32)
    out_ref[...] = jnp.maximum(acts + benc_ref[...], 0.0)


_encode = pl.pallas_call(
    _enc_body,
    grid=(_DICT // _BN_E, _BATCH // _BM_E),
    in_specs=[
        pl.BlockSpec((_BM_E, _ACT_DIM), lambda n, m: (m, 0)),
        pl.BlockSpec((_ACT_DIM, _BN_E), lambda n, m: (0, n)),
        pl.BlockSpec((1, _BN_E), lambda n, m: (0, n)),
        pl.BlockSpec((1, _ACT_DIM), lambda n, m: (0, 0)),
    ],
    out_specs=pl.BlockSpec((_BM_E, _BN_E), lambda n, m: (m, n)),
    out_shape=jax.ShapeDtypeStruct((_BATCH, _DICT), jnp.float32),
)


_NB1 = 32768
_NB2 = 65536
_NTILES = 32
_CHUNK = 16384
_ROWS_PER_TILE = _BATCH // _NTILES

@functools.lru_cache(maxsize=None)
def _sc_mesh():
    return plsc.VectorSubcoreMesh(core_axis_name="c", subcore_axis_name="s")


def _make_hist_body(pass2):
    nbins = _NB2 if pass2 else _NB1

    def body(*args):
        if pass2:
            acts_hbm, h_hbm, out_hbm, buf0, buf1, hist, hv, sem0, sem1 = args
        else:
            acts_hbm, out_hbm, buf0, buf1, hist, sem0, sem1 = args
        wid = lax.axis_index("s") * 2 + lax.axis_index("c")

        z = jnp.zeros((16,), jnp.float32)

        @plsc.parallel_loop(0, nbins // 16, unroll=8)
        def _(i):
            hist[pl.ds(i * 16, 16)] = z

        if pass2:
            pltpu.sync_copy(h_hbm, hv)
            hvec = hv[...]
        base = wid * _ROWS_PER_TILE
        ones = jnp.ones((16,), jnp.float32)
        sh16 = jnp.full((16,), 16, jnp.int32)
        lowm = jnp.full((16,), 0xFFFF, jnp.int32)

        def process(buf):
            @plsc.parallel_loop(0, _CHUNK // 16, unroll=8)
            def _(i):
                v = buf[pl.ds(i * 16, 16)]
                bits = plsc.bitcast(v, jnp.int32)
                if pass2:
                    hi = lax.shift_right_logical(bits, sh16)
                    m = hi == hvec
                    idx = jnp.bitwise_and(bits, lowm)
                else:
                    idx = lax.shift_right_logical(bits, sh16)
                    m = bits != 0
                plsc.addupdate_scatter(hist, [idx], ones, mask=m)

        def start(ci, buf, sem):
            pltpu.async_copy(acts_hbm.at[base + ci], buf, sem)

        def wait(buf, sem):
            pltpu.make_async_copy(acts_hbm.at[base], buf, sem).wait()

        start(0, buf0, sem0)

        def pair(i, c):
            a = 2 * i
            wait(buf0, sem0)
            start(a + 1, buf1, sem1)
            process(buf0)
            wait(buf1, sem1)

            @pl.when(a + 2 < _ROWS_PER_TILE)
            def _():
                start(a + 2, buf0, sem0)

            process(buf1)
            return c

        lax.fori_loop(0, _ROWS_PER_TILE // 2, pair, 0)
        pltpu.sync_copy(hist, out_hbm.at[wid])

    return body


_hist1_body = _make_hist_body(False)
_hist2_body = _make_hist_body(True)


@functools.lru_cache(maxsize=None)
def _hist1():
    return pl.kernel(
        _hist1_body,
        out_type=jax.ShapeDtypeStruct((_NTILES, _NB1), jnp.float32),
        mesh=_sc_mesh(),
        compiler_params=pltpu.CompilerParams(needs_layout_passes=False),
        scratch_types=[
            pltpu.VMEM((_CHUNK,), jnp.float32),
            pltpu.VMEM((_CHUNK,), jnp.float32),
            pltpu.VMEM((_NB1,), jnp.float32),
            pltpu.SemaphoreType.DMA,
            pltpu.SemaphoreType.DMA,
        ],
    )


@functools.lru_cache(maxsize=None)
def _hist2():
    return pl.kernel(
        _hist2_body,
        out_type=jax.ShapeDtypeStruct((_NTILES, _NB2), jnp.float32),
        mesh=_sc_mesh(),
        compiler_params=pltpu.CompilerParams(needs_layout_passes=False),
        scratch_types=[
            pltpu.VMEM((_CHUNK,), jnp.float32),
            pltpu.VMEM((_CHUNK,), jnp.float32),
            pltpu.VMEM((_NB2,), jnp.float32),
            pltpu.VMEM((16,), jnp.int32),
            pltpu.SemaphoreType.DMA,
            pltpu.SemaphoreType.DMA,
        ],
    )


def _above_counts(Hs, nrows):
    rs = jnp.sum(Hs, axis=1, keepdims=True)
    ii = lax.broadcasted_iota(jnp.int32, (nrows, nrows), 0)
    jj = lax.broadcasted_iota(jnp.int32, (nrows, nrows), 1)
    U = (jj > ii).astype(jnp.float32)
    r_after = jnp.dot(U, rs, preferred_element_type=jnp.float32,
                      precision=lax.Precision.HIGHEST)
    aa = lax.broadcasted_iota(jnp.int32, (128, 128), 0)
    bb = lax.broadcasted_iota(jnp.int32, (128, 128), 1)
    V = (aa > bb).astype(jnp.float32)
    w_after = jnp.dot(Hs, V, preferred_element_type=jnp.float32,
                      precision=lax.Precision.HIGHEST)
    return r_after + w_after


def _flat_iota(nrows):
    fi = (lax.broadcasted_iota(jnp.int32, (nrows, 128), 0) * 128
          + lax.broadcasted_iota(jnp.int32, (nrows, 128), 1))
    return fi.astype(jnp.float32)


def _sel1_body(hall_ref, out_ref):
    Hs = jnp.sum(hall_ref[...], axis=0)
    C = _above_counts(Hs, 256)
    kf = float(_KSEL)
    cond = jnp.logical_and(C < kf, C + Hs >= kf)
    h_f = jnp.sum(jnp.where(cond, _flat_iota(256), 0.0))
    a_f = jnp.sum(jnp.where(cond, C, 0.0))
    rio = lax.broadcasted_iota(jnp.int32, (8, 128), 0)
    out_ref[...] = jnp.where(
        rio == 0, h_f.astype(jnp.int32),
        jnp.where(rio == 1, a_f.astype(jnp.int32), 0))


_sel1 = pl.pallas_call(
    _sel1_body,
    out_shape=jax.ShapeDtypeStruct((8, 128), jnp.int32),
)


def _sel2_body(h2_ref, info_ref, out_ref):
    Hs = jnp.sum(h2_ref[...], axis=0)
    C = _above_counts(Hs, 512)
    h = info_ref[0, 0]
    a = info_ref[1, 0]
    kk = float(_KSEL) - a.astype(jnp.float32)
    cond = jnp.logical_and(C < kk, C + Hs >= kk)
    l_f = jnp.sum(jnp.where(cond, _flat_iota(512), 0.0))
    tau_bits = h * 65536 + l_f.astype(jnp.int32)
    out_ref[...] = lax.bitcast_convert_type(
        jnp.full((8, 128), tau_bits, jnp.int32), jnp.float32)


_sel2 = pl.pallas_call(
    _sel2_body,
    out_shape=jax.ShapeDtypeStruct((8, 128), jnp.float32),
)


_BM_D = 1024
_BK_D = 1024


def _dec_body(tau_ref, acts_ref, w_ref, bdec_ref, out_ref):
    kb = pl.program_id(1)
    tau = tau_ref[0, 0]
    a = acts_ref[...]
    a = jnp.where(a >= tau, a, 0.0).astype(jnp.bfloat16)
    part = jnp.dot(a, w_ref[...], preferred_element_type=jnp.float32)

    @pl.when(kb == 0)
    def _():
        out_ref[...] = part + bdec_ref[...]

    @pl.when(kb > 0)
    def _():
        out_ref[...] += part


_decode = pl.pallas_call(
    _dec_body,
    grid=(_BATCH // _BM_D, _DICT // _BK_D),
    in_specs=[
        pl.BlockSpec((8, 128), lambda m, k: (0, 0)),
        pl.BlockSpec((_BM_D, _BK_D), lambda m, k: (m, k)),
        pl.BlockSpec((_BK_D, _ACT_DIM), lambda m, k: (k, 0)),
        pl.BlockSpec((1, _ACT_DIM), lambda m, k: (0, 0)),
    ],
    out_specs=pl.BlockSpec((_BM_D, _ACT_DIM), lambda m, k: (m, 0)),
    out_shape=jax.ShapeDtypeStruct((_BATCH, _ACT_DIM), jnp.float32),
)


def kernel(x, W_enc, b_enc, W_dec, b_dec):
    acts = _encode(x, W_enc, b_enc.reshape(1, -1), b_dec.reshape(1, -1))
    h1 = _hist1()(acts)
    info = _sel1(h1.reshape(_NTILES, 256, 128))
    h_arr = info[0, :16]
    h2 = _hist2()(acts, h_arr)
    tau = _sel2(h2.reshape(_NTILES, 512, 128), info)
    return _decode(tau, acts, W_dec.astype(jnp.bfloat16), b_dec.reshape(1, -1))


# speedup vs baseline: 76.6229x; 1.0455x over previous
import functools

import jax
import jax.numpy as jnp
from jax import lax
from jax.experimental import pallas as pl
from jax.experimental.pallas import tpu as pltpu
from jax.experimental.pallas import tpu_sc as plsc

_ACT_DIM = 2048
_DICT = 16384
_BATCH = 2048
_KSEL = 64 * _BATCH
_NTOT = _BATCH * _DICT


_HALF = _BATCH // 2
_BM_E = 1024
_BN_E = 1024


def _enc_body(x_ref, w_ref, benc_ref, bdec_ref, out_ref):
    xb = x_ref[...] - bdec_ref[...]
    acts = jnp.dot(xb, w_ref[...], preferred_element_type=jnp.float32)
    out_ref[...] = jnp.maximum(acts + benc_ref[...], 0.0)


_encode = pl.pallas_call(
    _enc_body,
    grid=(_DICT // _BN_E, _HALF // _BM_E),
    in_specs=[
        pl.BlockSpec((_BM_E, _ACT_DIM), lambda n, m: (m, 0)),
        pl.BlockSpec((_ACT_DIM, _BN_E), lambda n, m: (0, n)),
        pl.BlockSpec((1, _BN_E), lambda n, m: (0, n)),
        pl.BlockSpec((1, _ACT_DIM), lambda n, m: (0, 0)),
    ],
    out_specs=pl.BlockSpec((_BM_E, _BN_E), lambda n, m: (m, n)),
    out_shape=jax.ShapeDtypeStruct((_HALF, _DICT), jnp.float32),
)


_NB1 = 32768
_NB2 = 65536
_NTILES = 32
_CHUNK = 16384
_ROWS_PER_TILE = _HALF // _NTILES

@functools.lru_cache(maxsize=None)
def _sc_mesh():
    return plsc.VectorSubcoreMesh(core_axis_name="c", subcore_axis_name="s")


def _make_hist_body(pass2):
    nbins = _NB2 if pass2 else _NB1

    def body(*args):
        if pass2:
            acts_hbm, h_hbm, out_hbm, buf0, buf1, hist, hv, sem0, sem1 = args
        else:
            acts_hbm, out_hbm, buf0, buf1, hist, sem0, sem1 = args
        wid = lax.axis_index("s") * 2 + lax.axis_index("c")

        z = jnp.zeros((16,), jnp.float32)

        @plsc.parallel_loop(0, nbins // 16, unroll=8)
        def _(i):
            hist[pl.ds(i * 16, 16)] = z

        if pass2:
            pltpu.sync_copy(h_hbm, hv)
            hvec = hv[...]
        base = wid * _ROWS_PER_TILE
        ones = jnp.ones((16,), jnp.float32)
        sh16 = jnp.full((16,), 16, jnp.int32)
        lowm = jnp.full((16,), 0xFFFF, jnp.int32)

        def process(buf):
            @plsc.parallel_loop(0, _CHUNK // 16, unroll=8)
            def _(i):
                v = buf[pl.ds(i * 16, 16)]
                bits = plsc.bitcast(v, jnp.int32)
                if pass2:
                    hi = lax.shift_right_logical(bits, sh16)
                    m = hi == hvec
                    idx = jnp.bitwise_and(bits, lowm)
                else:
                    idx = lax.shift_right_logical(bits, sh16)
                    m = bits != 0
                plsc.addupdate_scatter(hist, [idx], ones, mask=m)

        def start(ci, buf, sem):
            pltpu.async_copy(acts_hbm.at[base + ci], buf, sem)

        def wait(buf, sem):
            pltpu.make_async_copy(acts_hbm.at[base], buf, sem).wait()

        start(0, buf0, sem0)

        def pair(i, c):
            a = 2 * i
            wait(buf0, sem0)
            start(a + 1, buf1, sem1)
            process(buf0)
            wait(buf1, sem1)

            @pl.when(a + 2 < _ROWS_PER_TILE)
            def _():
                start(a + 2, buf0, sem0)

            process(buf1)
            return c

        lax.fori_loop(0, _ROWS_PER_TILE // 2, pair, 0)
        pltpu.sync_copy(hist, out_hbm.at[wid])

    return body


_hist1_body = _make_hist_body(False)
_hist2_body = _make_hist_body(True)


@functools.lru_cache(maxsize=None)
def _hist1():
    return pl.kernel(
        _hist1_body,
        out_type=jax.ShapeDtypeStruct((_NTILES, _NB1), jnp.float32),
        mesh=_sc_mesh(),
        compiler_params=pltpu.CompilerParams(needs_layout_passes=False),
        scratch_types=[
            pltpu.VMEM((_CHUNK,), jnp.float32),
            pltpu.VMEM((_CHUNK,), jnp.float32),
            pltpu.VMEM((_NB1,), jnp.float32),
            pltpu.SemaphoreType.DMA,
            pltpu.SemaphoreType.DMA,
        ],
    )


@functools.lru_cache(maxsize=None)
def _hist2():
    return pl.kernel(
        _hist2_body,
        out_type=jax.ShapeDtypeStruct((_NTILES, _NB2), jnp.float32),
        mesh=_sc_mesh(),
        compiler_params=pltpu.CompilerParams(needs_layout_passes=False),
        scratch_types=[
            pltpu.VMEM((_CHUNK,), jnp.float32),
            pltpu.VMEM((_CHUNK,), jnp.float32),
            pltpu.VMEM((_NB2,), jnp.float32),
            pltpu.VMEM((16,), jnp.int32),
            pltpu.SemaphoreType.DMA,
            pltpu.SemaphoreType.DMA,
        ],
    )


def _above_counts(Hs, nrows):
    rs = jnp.sum(Hs, axis=1, keepdims=True)
    ii = lax.broadcasted_iota(jnp.int32, (nrows, nrows), 0)
    jj = lax.broadcasted_iota(jnp.int32, (nrows, nrows), 1)
    U = (jj > ii).astype(jnp.float32)
    r_after = jnp.dot(U, rs, preferred_element_type=jnp.float32,
                      precision=lax.Precision.HIGHEST)
    aa = lax.broadcasted_iota(jnp.int32, (128, 128), 0)
    bb = lax.broadcasted_iota(jnp.int32, (128, 128), 1)
    V = (aa > bb).astype(jnp.float32)
    w_after = jnp.dot(Hs, V, preferred_element_type=jnp.float32,
                      precision=lax.Precision.HIGHEST)
    return r_after + w_after


def _flat_iota(nrows):
    fi = (lax.broadcasted_iota(jnp.int32, (nrows, 128), 0) * 128
          + lax.broadcasted_iota(jnp.int32, (nrows, 128), 1))
    return fi.astype(jnp.float32)


def _sel1_body(ha_ref, hb_ref, out_ref):
    Hs = jnp.sum(ha_ref[...], axis=0) + jnp.sum(hb_ref[...], axis=0)
    C = _above_counts(Hs, 256)
    kf = float(_KSEL)
    cond = jnp.logical_and(C < kf, C + Hs >= kf)
    h_f = jnp.sum(jnp.where(cond, _flat_iota(256), 0.0))
    a_f = jnp.sum(jnp.where(cond, C, 0.0))
    rio = lax.broadcasted_iota(jnp.int32, (8, 128), 0)
    out_ref[...] = jnp.where(
        rio == 0, h_f.astype(jnp.int32),
        jnp.where(rio == 1, a_f.astype(jnp.int32), 0))


_sel1 = pl.pallas_call(
    _sel1_body,
    out_shape=jax.ShapeDtypeStruct((8, 128), jnp.int32),
)


def _sel2_body(ha_ref, hb_ref, info_ref, out_ref):
    Hs = jnp.sum(ha_ref[...], axis=0) + jnp.sum(hb_ref[...], axis=0)
    C = _above_counts(Hs, 512)
    h = info_ref[0, 0]
    a = info_ref[1, 0]
    kk = float(_KSEL) - a.astype(jnp.float32)
    cond = jnp.logical_and(C < kk, C + Hs >= kk)
    l_f = jnp.sum(jnp.where(cond, _flat_iota(512), 0.0))
    tau_bits = h * 65536 + l_f.astype(jnp.int32)
    out_ref[...] = lax.bitcast_convert_type(
        jnp.full((8, 128), tau_bits, jnp.int32), jnp.float32)


_sel2 = pl.pallas_call(
    _sel2_body,
    out_shape=jax.ShapeDtypeStruct((8, 128), jnp.float32),
)


_BM_D = 1024
_BK_D = 1024


def _dec_body(tau_ref, acts_ref, w_ref, bdec_ref, out_ref):
    kb = pl.program_id(1)
    tau = tau_ref[0, 0]
    a = acts_ref[...]
    a = jnp.where(a >= tau, a, 0.0).astype(jnp.bfloat16)
    part = jnp.dot(a, w_ref[...], preferred_element_type=jnp.float32)

    @pl.when(kb == 0)
    def _():
        out_ref[...] = part + bdec_ref[...]

    @pl.when(kb > 0)
    def _():
        out_ref[...] += part


_decode = pl.pallas_call(
    _dec_body,
    grid=(_HALF // _BM_D, _DICT // _BK_D),
    in_specs=[
        pl.BlockSpec((8, 128), lambda m, k: (0, 0)),
        pl.BlockSpec((_BM_D, _BK_D), lambda m, k: (m, k)),
        pl.BlockSpec((_BK_D, _ACT_DIM), lambda m, k: (k, 0)),
        pl.BlockSpec((1, _ACT_DIM), lambda m, k: (0, 0)),
    ],
    out_specs=pl.BlockSpec((_BM_D, _ACT_DIM), lambda m, k: (m, 0)),
    out_shape=jax.ShapeDtypeStruct((_HALF, _ACT_DIM), jnp.float32),
)


def kernel(x, W_enc, b_enc, W_dec, b_dec):
    be = b_enc.reshape(1, -1)
    bd = b_dec.reshape(1, -1)
    acts_a = _encode(x[:_HALF], W_enc, be, bd)
    acts_b = _encode(x[_HALF:], W_enc, be, bd)
    h1a = _hist1()(acts_a)
    h1b = _hist1()(acts_b)
    info = _sel1(h1a.reshape(_NTILES, 256, 128),
                 h1b.reshape(_NTILES, 256, 128))
    h_arr = info[0, :16]
    h2a = _hist2()(acts_a, h_arr)
    h2b = _hist2()(acts_b, h_arr)
    tau = _sel2(h2a.reshape(_NTILES, 512, 128),
                h2b.reshape(_NTILES, 512, 128), info)
    w_bf = W_dec.astype(jnp.bfloat16)
    xh_a = _decode(tau, acts_a, w_bf, bd)
    xh_b = _decode(tau, acts_b, w_bf, bd)
    return jnp.concatenate([xh_a, xh_b], axis=0)


# speedup vs baseline: 82.2540x; 1.0735x over previous
import functools

import jax
import jax.numpy as jnp
from jax import lax
from jax.experimental import pallas as pl
from jax.experimental.pallas import tpu as pltpu
from jax.experimental.pallas import tpu_sc as plsc

_ACT_DIM = 2048
_DICT = 16384
_BATCH = 2048
_KSEL = 64 * _BATCH
_NTOT = _BATCH * _DICT


_HALF = _BATCH // 2
_BM_E = 1024
_BN_E = 1024


def _enc_body(x_ref, w_ref, benc_ref, bdec_ref, out_ref):
    xb = x_ref[...] - bdec_ref[...]
    acts = jnp.dot(xb, w_ref[...], preferred_element_type=jnp.float32)
    out_ref[...] = jnp.maximum(acts + benc_ref[...], 0.0)


_encode = pl.pallas_call(
    _enc_body,
    grid=(_DICT // _BN_E, _HALF // _BM_E),
    in_specs=[
        pl.BlockSpec((_BM_E, _ACT_DIM), lambda n, m: (m, 0)),
        pl.BlockSpec((_ACT_DIM, _BN_E), lambda n, m: (0, n)),
        pl.BlockSpec((1, _BN_E), lambda n, m: (0, n)),
        pl.BlockSpec((1, _ACT_DIM), lambda n, m: (0, 0)),
    ],
    out_specs=pl.BlockSpec((_BM_E, _BN_E), lambda n, m: (m, n)),
    out_shape=jax.ShapeDtypeStruct((_HALF, _DICT), jnp.float32),
)


_NB1 = 32768
_NB2 = 65536
_NTILES = 32
_CHUNK = 16384
_ROWS_PER_TILE = _HALF // _NTILES

@functools.lru_cache(maxsize=None)
def _sc_mesh():
    return plsc.VectorSubcoreMesh(core_axis_name="c", subcore_axis_name="s")


def _make_hist_body(pass2):
    nbins = _NB2 if pass2 else _NB1

    def body(*args):
        if pass2:
            acts_hbm, h_hbm, out_hbm, buf0, buf1, hist, hv, sem0, sem1 = args
        else:
            acts_hbm, out_hbm, buf0, buf1, hist, sem0, sem1 = args
        wid = lax.axis_index("s") * 2 + lax.axis_index("c")

        z = jnp.zeros((16,), jnp.float32)

        @plsc.parallel_loop(0, nbins // 16, unroll=8)
        def _(i):
            hist[pl.ds(i * 16, 16)] = z

        if pass2:
            pltpu.sync_copy(h_hbm, hv)
            hvec = hv[...]
        base = wid * _ROWS_PER_TILE
        ones = jnp.ones((16,), jnp.float32)
        sh16 = jnp.full((16,), 16, jnp.int32)
        lowm = jnp.full((16,), 0xFFFF, jnp.int32)

        def process(buf):
            @plsc.parallel_loop(0, _CHUNK // 16, unroll=8)
            def _(i):
                v = buf[pl.ds(i * 16, 16)]
                bits = plsc.bitcast(v, jnp.int32)
                if pass2:
                    hi = lax.shift_right_logical(bits, sh16)
                    m = hi == hvec
                    idx = jnp.bitwise_and(bits, lowm)
                else:
                    idx = lax.shift_right_logical(bits, sh16)
                    m = bits != 0
                plsc.addupdate_scatter(hist, [idx], ones, mask=m)

        def start(ci, buf, sem):
            pltpu.async_copy(acts_hbm.at[base + ci], buf, sem)

        def wait(buf, sem):
            pltpu.make_async_copy(acts_hbm.at[base], buf, sem).wait()

        start(0, buf0, sem0)

        def pair(i, c):
            a = 2 * i
            wait(buf0, sem0)
            start(a + 1, buf1, sem1)
            process(buf0)
            wait(buf1, sem1)

            @pl.when(a + 2 < _ROWS_PER_TILE)
            def _():
                start(a + 2, buf0, sem0)

            process(buf1)
            return c

        lax.fori_loop(0, _ROWS_PER_TILE // 2, pair, 0)
        pltpu.sync_copy(hist, out_hbm.at[wid])

    return body


_hist1_body = _make_hist_body(False)
_hist2_body = _make_hist_body(True)


@functools.lru_cache(maxsize=None)
def _hist1():
    return pl.kernel(
        _hist1_body,
        out_type=jax.ShapeDtypeStruct((_NTILES, _NB1), jnp.float32),
        mesh=_sc_mesh(),
        compiler_params=pltpu.CompilerParams(needs_layout_passes=False),
        scratch_types=[
            pltpu.VMEM((_CHUNK,), jnp.float32),
            pltpu.VMEM((_CHUNK,), jnp.float32),
            pltpu.VMEM((_NB1,), jnp.float32),
            pltpu.SemaphoreType.DMA,
            pltpu.SemaphoreType.DMA,
        ],
    )


@functools.lru_cache(maxsize=None)
def _hist2():
    return pl.kernel(
        _hist2_body,
        out_type=jax.ShapeDtypeStruct((_NTILES, _NB2), jnp.float32),
        mesh=_sc_mesh(),
        compiler_params=pltpu.CompilerParams(needs_layout_passes=False),
        scratch_types=[
            pltpu.VMEM((_CHUNK,), jnp.float32),
            pltpu.VMEM((_CHUNK,), jnp.float32),
            pltpu.VMEM((_NB2,), jnp.float32),
            pltpu.VMEM((16,), jnp.int32),
            pltpu.SemaphoreType.DMA,
            pltpu.SemaphoreType.DMA,
        ],
    )


def _above_counts(Hs, nrows):
    rs = jnp.sum(Hs, axis=1, keepdims=True)
    ii = lax.broadcasted_iota(jnp.int32, (nrows, nrows), 0)
    jj = lax.broadcasted_iota(jnp.int32, (nrows, nrows), 1)
    U = (jj > ii).astype(jnp.float32)
    r_after = jnp.dot(U, rs, preferred_element_type=jnp.float32,
                      precision=lax.Precision.HIGHEST)
    aa = lax.broadcasted_iota(jnp.int32, (128, 128), 0)
    bb = lax.broadcasted_iota(jnp.int32, (128, 128), 1)
    V = (aa > bb).astype(jnp.float32)
    w_after = jnp.dot(Hs, V, preferred_element_type=jnp.float32,
                      precision=lax.Precision.HIGHEST)
    return r_after + w_after


def _flat_iota(nrows):
    fi = (lax.broadcasted_iota(jnp.int32, (nrows, 128), 0) * 128
          + lax.broadcasted_iota(jnp.int32, (nrows, 128), 1))
    return fi.astype(jnp.float32)


def _sel1_body(ha_ref, hb_ref, out_ref):
    Hs = jnp.sum(ha_ref[...], axis=0) + jnp.sum(hb_ref[...], axis=0)
    C = _above_counts(Hs, 256)
    kf = float(_KSEL)
    cond = jnp.logical_and(C < kf, C + Hs >= kf)
    h_f = jnp.sum(jnp.where(cond, _flat_iota(256), 0.0))
    a_f = jnp.sum(jnp.where(cond, C, 0.0))
    rio = lax.broadcasted_iota(jnp.int32, (8, 128), 0)
    out_ref[...] = jnp.where(
        rio == 0, h_f.astype(jnp.int32),
        jnp.where(rio == 1, a_f.astype(jnp.int32), 0))


_sel1 = pl.pallas_call(
    _sel1_body,
    out_shape=jax.ShapeDtypeStruct((8, 128), jnp.int32),
)


def _sel2_body(ha_ref, hb_ref, info_ref, out_ref):
    Hs = jnp.sum(ha_ref[...], axis=0) + jnp.sum(hb_ref[...], axis=0)
    C = _above_counts(Hs, 512)
    h = info_ref[0, 0]
    a = info_ref[1, 0]
    kk = float(_KSEL) - a.astype(jnp.float32)
    cond = jnp.logical_and(C < kk, C + Hs >= kk)
    l_f = jnp.sum(jnp.where(cond, _flat_iota(512), 0.0))
    tau_bits = h * 65536 + l_f.astype(jnp.int32)
    out_ref[...] = lax.bitcast_convert_type(
        jnp.full((8, 128), tau_bits, jnp.int32), jnp.float32)


_sel2 = pl.pallas_call(
    _sel2_body,
    out_shape=jax.ShapeDtypeStruct((8, 128), jnp.float32),
)


_BM_D = 1024
_BK_D = 1024


def _dec_body(tau_ref, acts_ref, w_ref, bdec_ref, out_ref):
    kb = pl.program_id(1)
    tau = tau_ref[0, 0]
    a = acts_ref[...]
    a = jnp.where(a >= tau, a, 0.0).astype(jnp.bfloat16)
    w = w_ref[...].astype(jnp.bfloat16)
    part = jnp.dot(a, w, preferred_element_type=jnp.float32)

    @pl.when(kb == 0)
    def _():
        out_ref[...] = part + bdec_ref[...]

    @pl.when(kb > 0)
    def _():
        out_ref[...] += part


_decode = pl.pallas_call(
    _dec_body,
    grid=(_HALF // _BM_D, _DICT // _BK_D),
    in_specs=[
        pl.BlockSpec((8, 128), lambda m, k: (0, 0)),
        pl.BlockSpec((_BM_D, _BK_D), lambda m, k: (m, k)),
        pl.BlockSpec((_BK_D, _ACT_DIM), lambda m, k: (k, 0)),
        pl.BlockSpec((1, _ACT_DIM), lambda m, k: (0, 0)),
    ],
    out_specs=pl.BlockSpec((_BM_D, _ACT_DIM), lambda m, k: (m, 0)),
    out_shape=jax.ShapeDtypeStruct((_HALF, _ACT_DIM), jnp.float32),
)


def kernel(x, W_enc, b_enc, W_dec, b_dec):
    be = b_enc.reshape(1, -1)
    bd = b_dec.reshape(1, -1)
    acts_a = _encode(x[:_HALF], W_enc, be, bd)
    acts_b = _encode(x[_HALF:], W_enc, be, bd)
    h1a = _hist1()(acts_a)
    h1b = _hist1()(acts_b)
    info = _sel1(h1a.reshape(_NTILES, 256, 128),
                 h1b.reshape(_NTILES, 256, 128))
    h_arr = info[0, :16]
    h2a = _hist2()(acts_a, h_arr)
    h2b = _hist2()(acts_b, h_arr)
    tau = _sel2(h2a.reshape(_NTILES, 512, 128),
                h2b.reshape(_NTILES, 512, 128), info)
    xh_a = _decode(tau, acts_a, W_dec, bd)
    xh_b = _decode(tau, acts_b, W_dec, bd)
    return jnp.concatenate([xh_a, xh_b], axis=0)


# speedup vs baseline: 85.7347x; 1.0423x over previous
import functools

import jax
import jax.numpy as jnp
from jax import lax
from jax.experimental import pallas as pl
from jax.experimental.pallas import tpu as pltpu
from jax.experimental.pallas import tpu_sc as plsc

_ACT_DIM = 2048
_DICT = 16384
_BATCH = 2048
_KSEL = 64 * _BATCH
_NTOT = _BATCH * _DICT


_HALF = _BATCH // 2
_BM_E = 1024
_BN_E = 1024


def _enc_body(x_ref, w_ref, benc_ref, bdec_ref, out_ref):
    xb = x_ref[...] - bdec_ref[...]
    acts = jnp.dot(xb, w_ref[...], preferred_element_type=jnp.float32)
    out_ref[...] = jnp.maximum(acts + benc_ref[...], 0.0)


_encode = pl.pallas_call(
    _enc_body,
    grid=(_DICT // _BN_E, _HALF // _BM_E),
    in_specs=[
        pl.BlockSpec((_BM_E, _ACT_DIM), lambda n, m: (m, 0)),
        pl.BlockSpec((_ACT_DIM, _BN_E), lambda n, m: (0, n)),
        pl.BlockSpec((1, _BN_E), lambda n, m: (0, n)),
        pl.BlockSpec((1, _ACT_DIM), lambda n, m: (0, 0)),
    ],
    out_specs=pl.BlockSpec((_BM_E, _BN_E), lambda n, m: (m, n)),
    out_shape=jax.ShapeDtypeStruct((_HALF, _DICT), jnp.float32),
)


_NB1 = 32768
_NB2 = 65536
_NTILES = 32
_CHUNK = 16384
_ROWS_PER_TILE = _HALF // _NTILES

@functools.lru_cache(maxsize=None)
def _sc_mesh():
    return plsc.VectorSubcoreMesh(core_axis_name="c", subcore_axis_name="s")


def _make_hist_body(pass2):
    nbins = _NB2 if pass2 else _NB1

    def body(*args):
        if pass2:
            (acts_hbm, acts2_hbm, h_hbm, out_hbm,
             buf0, buf1, hist, hv, sem0, sem1) = args
        else:
            acts_hbm, out_hbm, buf0, buf1, hist, sem0, sem1 = args
        wid = lax.axis_index("s") * 2 + lax.axis_index("c")

        z = jnp.zeros((16,), jnp.float32)

        @plsc.parallel_loop(0, nbins // 16, unroll=8)
        def _(i):
            hist[pl.ds(i * 16, 16)] = z

        if pass2:
            pltpu.sync_copy(h_hbm, hv)
            hvec = hv[...]
        base = wid * _ROWS_PER_TILE
        ones = jnp.ones((16,), jnp.float32)
        sh16 = jnp.full((16,), 16, jnp.int32)
        lowm = jnp.full((16,), 0xFFFF, jnp.int32)

        def process(buf):
            @plsc.parallel_loop(0, _CHUNK // 16, unroll=8)
            def _(i):
                v = buf[pl.ds(i * 16, 16)]
                bits = plsc.bitcast(v, jnp.int32)
                if pass2:
                    hi = lax.shift_right_logical(bits, sh16)
                    m = hi == hvec
                    idx = jnp.bitwise_and(bits, lowm)
                else:
                    idx = lax.shift_right_logical(bits, sh16)
                    m = bits != 0
                plsc.addupdate_scatter(hist, [idx], ones, mask=m)

        def scan_rows(src):
            def start(ci, buf, sem):
                pltpu.async_copy(src.at[base + ci], buf, sem)

            def wait(buf, sem):
                pltpu.make_async_copy(src.at[base], buf, sem).wait()

            start(0, buf0, sem0)

            def pair(i, c):
                a = 2 * i
                wait(buf0, sem0)
                start(a + 1, buf1, sem1)
                process(buf0)
                wait(buf1, sem1)

                @pl.when(a + 2 < _ROWS_PER_TILE)
                def _():
                    start(a + 2, buf0, sem0)

                process(buf1)
                return c

            lax.fori_loop(0, _ROWS_PER_TILE // 2, pair, 0)

        scan_rows(acts_hbm)
        if pass2:
            scan_rows(acts2_hbm)
        pltpu.sync_copy(hist, out_hbm.at[wid])

    return body


_hist1_body = _make_hist_body(False)
_hist2_body = _make_hist_body(True)


@functools.lru_cache(maxsize=None)
def _hist1():
    return pl.kernel(
        _hist1_body,
        out_type=jax.ShapeDtypeStruct((_NTILES, _NB1), jnp.float32),
        mesh=_sc_mesh(),
        compiler_params=pltpu.CompilerParams(needs_layout_passes=False),
        scratch_types=[
            pltpu.VMEM((_CHUNK,), jnp.float32),
            pltpu.VMEM((_CHUNK,), jnp.float32),
            pltpu.VMEM((_NB1,), jnp.float32),
            pltpu.SemaphoreType.DMA,
            pltpu.SemaphoreType.DMA,
        ],
    )


@functools.lru_cache(maxsize=None)
def _hist2():
    return pl.kernel(
        _hist2_body,
        out_type=jax.ShapeDtypeStruct((_NTILES, _NB2), jnp.float32),
        mesh=_sc_mesh(),
        compiler_params=pltpu.CompilerParams(needs_layout_passes=False),
        scratch_types=[
            pltpu.VMEM((_CHUNK,), jnp.float32),
            pltpu.VMEM((_CHUNK,), jnp.float32),
            pltpu.VMEM((_NB2,), jnp.float32),
            pltpu.VMEM((16,), jnp.int32),
            pltpu.SemaphoreType.DMA,
            pltpu.SemaphoreType.DMA,
        ],
    )


def _above_counts(Hs, nrows):
    rs = jnp.sum(Hs, axis=1, keepdims=True)
    ii = lax.broadcasted_iota(jnp.int32, (nrows, nrows), 0)
    jj = lax.broadcasted_iota(jnp.int32, (nrows, nrows), 1)
    U = (jj > ii).astype(jnp.float32)
    r_after = jnp.dot(U, rs, preferred_element_type=jnp.float32,
                      precision=lax.Precision.HIGHEST)
    aa = lax.broadcasted_iota(jnp.int32, (128, 128), 0)
    bb = lax.broadcasted_iota(jnp.int32, (128, 128), 1)
    V = (aa > bb).astype(jnp.float32)
    w_after = jnp.dot(Hs, V, preferred_element_type=jnp.float32,
                      precision=lax.Precision.HIGHEST)
    return r_after + w_after


def _flat_iota(nrows):
    fi = (lax.broadcasted_iota(jnp.int32, (nrows, 128), 0) * 128
          + lax.broadcasted_iota(jnp.int32, (nrows, 128), 1))
    return fi.astype(jnp.float32)


def _sel1_body(ha_ref, hb_ref, out_ref):
    Hs = jnp.sum(ha_ref[...], axis=0) + jnp.sum(hb_ref[...], axis=0)
    C = _above_counts(Hs, 256)
    kf = float(_KSEL)
    cond = jnp.logical_and(C < kf, C + Hs >= kf)
    h_f = jnp.sum(jnp.where(cond, _flat_iota(256), 0.0))
    a_f = jnp.sum(jnp.where(cond, C, 0.0))
    rio = lax.broadcasted_iota(jnp.int32, (8, 128), 0)
    out_ref[...] = jnp.where(
        rio == 0, h_f.astype(jnp.int32),
        jnp.where(rio == 1, a_f.astype(jnp.int32), 0))


_sel1 = pl.pallas_call(
    _sel1_body,
    out_shape=jax.ShapeDtypeStruct((8, 128), jnp.int32),
)


def _sel2_body(h2_ref, info_ref, out_ref):
    Hs = jnp.sum(h2_ref[...], axis=0)
    C = _above_counts(Hs, 512)
    h = info_ref[0, 0]
    a = info_ref[1, 0]
    kk = float(_KSEL) - a.astype(jnp.float32)
    cond = jnp.logical_and(C < kk, C + Hs >= kk)
    l_f = jnp.sum(jnp.where(cond, _flat_iota(512), 0.0))
    tau_bits = h * 65536 + l_f.astype(jnp.int32)
    out_ref[...] = lax.bitcast_convert_type(
        jnp.full((8, 128), tau_bits, jnp.int32), jnp.float32)


_sel2 = pl.pallas_call(
    _sel2_body,
    out_shape=jax.ShapeDtypeStruct((8, 128), jnp.float32),
)


_BM_D = 1024
_BK_D = 1024


def _dec_body(tau_ref, acts_ref, w_ref, bdec_ref, out_ref):
    kb = pl.program_id(1)
    tau = tau_ref[0, 0]
    a = acts_ref[...]
    a = jnp.where(a >= tau, a, 0.0).astype(jnp.bfloat16)
    w = w_ref[...].astype(jnp.bfloat16)
    part = jnp.dot(a, w, preferred_element_type=jnp.float32)

    @pl.when(kb == 0)
    def _():
        out_ref[...] = part + bdec_ref[...]

    @pl.when(kb > 0)
    def _():
        out_ref[...] += part


_decode = pl.pallas_call(
    _dec_body,
    grid=(_HALF // _BM_D, _DICT // _BK_D),
    in_specs=[
        pl.BlockSpec((8, 128), lambda m, k: (0, 0)),
        pl.BlockSpec((_BM_D, _BK_D), lambda m, k: (m, k)),
        pl.BlockSpec((_BK_D, _ACT_DIM), lambda m, k: (k, 0)),
        pl.BlockSpec((1, _ACT_DIM), lambda m, k: (0, 0)),
    ],
    out_specs=pl.BlockSpec((_BM_D, _ACT_DIM), lambda m, k: (m, 0)),
    out_shape=jax.ShapeDtypeStruct((_HALF, _ACT_DIM), jnp.float32),
)


def kernel(x, W_enc, b_enc, W_dec, b_dec):
    be = b_enc.reshape(1, -1)
    bd = b_dec.reshape(1, -1)
    acts_a = _encode(x[:_HALF], W_enc, be, bd)
    acts_b = _encode(x[_HALF:], W_enc, be, bd)
    h1a = _hist1()(acts_a)
    h1b = _hist1()(acts_b)
    info = _sel1(h1a.reshape(_NTILES, 256, 128),
                 h1b.reshape(_NTILES, 256, 128))
    h_arr = info[0, :16]
    h2 = _hist2()(acts_a, acts_b, h_arr)
    tau = _sel2(h2.reshape(_NTILES, 512, 128), info)
    xh_a = _decode(tau, acts_a, W_dec, bd)
    xh_b = _decode(tau, acts_b, W_dec, bd)
    return jnp.concatenate([xh_a, xh_b], axis=0)


# speedup vs baseline: 85.7475x; 1.0001x over previous
import functools

import jax
import jax.numpy as jnp
from jax import lax
from jax.experimental import pallas as pl
from jax.experimental.pallas import tpu as pltpu
from jax.experimental.pallas import tpu_sc as plsc

_ACT_DIM = 2048
_DICT = 16384
_BATCH = 2048
_KSEL = 64 * _BATCH
_NTOT = _BATCH * _DICT


_HALF = _BATCH // 2
_BM_E = 1024
_BN_E = 1024


def _enc_body(x_ref, w_ref, benc_ref, bdec_ref, out_ref):
    xb = (x_ref[...] - bdec_ref[...]).astype(jnp.bfloat16)
    w = w_ref[...].astype(jnp.bfloat16)
    acts = jnp.dot(xb, w, preferred_element_type=jnp.float32)
    out_ref[...] = jnp.maximum(acts + benc_ref[...], 0.0)


_encode = pl.pallas_call(
    _enc_body,
    grid=(_DICT // _BN_E, _HALF // _BM_E),
    in_specs=[
        pl.BlockSpec((_BM_E, _ACT_DIM), lambda n, m: (m, 0)),
        pl.BlockSpec((_ACT_DIM, _BN_E), lambda n, m: (0, n)),
        pl.BlockSpec((1, _BN_E), lambda n, m: (0, n)),
        pl.BlockSpec((1, _ACT_DIM), lambda n, m: (0, 0)),
    ],
    out_specs=pl.BlockSpec((_BM_E, _BN_E), lambda n, m: (m, n)),
    out_shape=jax.ShapeDtypeStruct((_HALF, _DICT), jnp.float32),
)


_NB1 = 32768
_NB2 = 65536
_NTILES = 32
_CHUNK = 16384
_ROWS_PER_TILE = _HALF // _NTILES

@functools.lru_cache(maxsize=None)
def _sc_mesh():
    return plsc.VectorSubcoreMesh(core_axis_name="c", subcore_axis_name="s")


def _make_hist_body(pass2):
    nbins = _NB2 if pass2 else _NB1

    def body(*args):
        if pass2:
            (acts_hbm, acts2_hbm, h_hbm, out_hbm,
             buf0, buf1, hist, hv, sem0, sem1) = args
        else:
            acts_hbm, out_hbm, buf0, buf1, hist, sem0, sem1 = args
        wid = lax.axis_index("s") * 2 + lax.axis_index("c")

        z = jnp.zeros((16,), jnp.float32)

        @plsc.parallel_loop(0, nbins // 16, unroll=8)
        def _(i):
            hist[pl.ds(i * 16, 16)] = z

        if pass2:
            pltpu.sync_copy(h_hbm, hv)
            hvec = hv[...]
        base = wid * _ROWS_PER_TILE
        ones = jnp.ones((16,), jnp.float32)
        sh16 = jnp.full((16,), 16, jnp.int32)
        lowm = jnp.full((16,), 0xFFFF, jnp.int32)

        def process(buf):
            @plsc.parallel_loop(0, _CHUNK // 16, unroll=8)
            def _(i):
                v = buf[pl.ds(i * 16, 16)]
                bits = plsc.bitcast(v, jnp.int32)
                if pass2:
                    hi = lax.shift_right_logical(bits, sh16)
                    m = hi == hvec
                    idx = jnp.bitwise_and(bits, lowm)
                else:
                    idx = lax.shift_right_logical(bits, sh16)
                    m = bits != 0
                plsc.addupdate_scatter(hist, [idx], ones, mask=m)

        def scan_rows(src):
            def start(ci, buf, sem):
                pltpu.async_copy(src.at[base + ci], buf, sem)

            def wait(buf, sem):
                pltpu.make_async_copy(src.at[base], buf, sem).wait()

            start(0, buf0, sem0)

            def pair(i, c):
                a = 2 * i
                wait(buf0, sem0)
                start(a + 1, buf1, sem1)
                process(buf0)
                wait(buf1, sem1)

                @pl.when(a + 2 < _ROWS_PER_TILE)
                def _():
                    start(a + 2, buf0, sem0)

                process(buf1)
                return c

            lax.fori_loop(0, _ROWS_PER_TILE // 2, pair, 0)

        scan_rows(acts_hbm)
        if pass2:
            scan_rows(acts2_hbm)
        pltpu.sync_copy(hist, out_hbm.at[wid])

    return body


_hist1_body = _make_hist_body(False)
_hist2_body = _make_hist_body(True)


@functools.lru_cache(maxsize=None)
def _hist1():
    return pl.kernel(
        _hist1_body,
        out_type=jax.ShapeDtypeStruct((_NTILES, _NB1), jnp.float32),
        mesh=_sc_mesh(),
        compiler_params=pltpu.CompilerParams(needs_layout_passes=False),
        scratch_types=[
            pltpu.VMEM((_CHUNK,), jnp.float32),
            pltpu.VMEM((_CHUNK,), jnp.float32),
            pltpu.VMEM((_NB1,), jnp.float32),
            pltpu.SemaphoreType.DMA,
            pltpu.SemaphoreType.DMA,
        ],
    )


@functools.lru_cache(maxsize=None)
def _hist2():
    return pl.kernel(
        _hist2_body,
        out_type=jax.ShapeDtypeStruct((_NTILES, _NB2), jnp.float32),
        mesh=_sc_mesh(),
        compiler_params=pltpu.CompilerParams(needs_layout_passes=False),
        scratch_types=[
            pltpu.VMEM((_CHUNK,), jnp.float32),
            pltpu.VMEM((_CHUNK,), jnp.float32),
            pltpu.VMEM((_NB2,), jnp.float32),
            pltpu.VMEM((16,), jnp.int32),
            pltpu.SemaphoreType.DMA,
            pltpu.SemaphoreType.DMA,
        ],
    )


def _above_counts(Hs, nrows):
    rs = jnp.sum(Hs, axis=1, keepdims=True)
    ii = lax.broadcasted_iota(jnp.int32, (nrows, nrows), 0)
    jj = lax.broadcasted_iota(jnp.int32, (nrows, nrows), 1)
    U = (jj > ii).astype(jnp.float32)
    r_after = jnp.dot(U, rs, preferred_element_type=jnp.float32,
                      precision=lax.Precision.HIGHEST)
    aa = lax.broadcasted_iota(jnp.int32, (128, 128), 0)
    bb = lax.broadcasted_iota(jnp.int32, (128, 128), 1)
    V = (aa > bb).astype(jnp.float32)
    w_after = jnp.dot(Hs, V, preferred_element_type=jnp.float32,
                      precision=lax.Precision.HIGHEST)
    return r_after + w_after


def _flat_iota(nrows):
    fi = (lax.broadcasted_iota(jnp.int32, (nrows, 128), 0) * 128
          + lax.broadcasted_iota(jnp.int32, (nrows, 128), 1))
    return fi.astype(jnp.float32)


def _sel1_body(ha_ref, hb_ref, out_ref):
    Hs = jnp.sum(ha_ref[...], axis=0) + jnp.sum(hb_ref[...], axis=0)
    C = _above_counts(Hs, 256)
    kf = float(_KSEL)
    cond = jnp.logical_and(C < kf, C + Hs >= kf)
    h_f = jnp.sum(jnp.where(cond, _flat_iota(256), 0.0))
    a_f = jnp.sum(jnp.where(cond, C, 0.0))
    rio = lax.broadcasted_iota(jnp.int32, (8, 128), 0)
    out_ref[...] = jnp.where(
        rio == 0, h_f.astype(jnp.int32),
        jnp.where(rio == 1, a_f.astype(jnp.int32), 0))


_sel1 = pl.pallas_call(
    _sel1_body,
    out_shape=jax.ShapeDtypeStruct((8, 128), jnp.int32),
)


def _sel2_body(h2_ref, info_ref, out_ref):
    Hs = jnp.sum(h2_ref[...], axis=0)
    C = _above_counts(Hs, 512)
    h = info_ref[0, 0]
    a = info_ref[1, 0]
    kk = float(_KSEL) - a.astype(jnp.float32)
    cond = jnp.logical_and(C < kk, C + Hs >= kk)
    l_f = jnp.sum(jnp.where(cond, _flat_iota(512), 0.0))
    tau_bits = h * 65536 + l_f.astype(jnp.int32)
    out_ref[...] = lax.bitcast_convert_type(
        jnp.full((8, 128), tau_bits, jnp.int32), jnp.float32)


_sel2 = pl.pallas_call(
    _sel2_body,
    out_shape=jax.ShapeDtypeStruct((8, 128), jnp.float32),
)


_BM_D = 1024
_BK_D = 1024


def _dec_body(tau_ref, acts_ref, w_ref, bdec_ref, out_ref):
    kb = pl.program_id(1)
    tau = tau_ref[0, 0]
    a = acts_ref[...]
    a = jnp.where(a >= tau, a, 0.0).astype(jnp.bfloat16)
    w = w_ref[...].astype(jnp.bfloat16)
    part = jnp.dot(a, w, preferred_element_type=jnp.float32)

    @pl.when(kb == 0)
    def _():
        out_ref[...] = part + bdec_ref[...]

    @pl.when(kb > 0)
    def _():
        out_ref[...] += part


_decode = pl.pallas_call(
    _dec_body,
    grid=(_HALF // _BM_D, _DICT // _BK_D),
    in_specs=[
        pl.BlockSpec((8, 128), lambda m, k: (0, 0)),
        pl.BlockSpec((_BM_D, _BK_D), lambda m, k: (m, k)),
        pl.BlockSpec((_BK_D, _ACT_DIM), lambda m, k: (k, 0)),
        pl.BlockSpec((1, _ACT_DIM), lambda m, k: (0, 0)),
    ],
    out_specs=pl.BlockSpec((_BM_D, _ACT_DIM), lambda m, k: (m, 0)),
    out_shape=jax.ShapeDtypeStruct((_HALF, _ACT_DIM), jnp.float32),
)


def kernel(x, W_enc, b_enc, W_dec, b_dec):
    be = b_enc.reshape(1, -1)
    bd = b_dec.reshape(1, -1)
    acts_a = _encode(x[:_HALF], W_enc, be, bd)
    acts_b = _encode(x[_HALF:], W_enc, be, bd)
    h1a = _hist1()(acts_a)
    h1b = _hist1()(acts_b)
    info = _sel1(h1a.reshape(_NTILES, 256, 128),
                 h1b.reshape(_NTILES, 256, 128))
    h_arr = info[0, :16]
    h2 = _hist2()(acts_a, acts_b, h_arr)
    tau = _sel2(h2.reshape(_NTILES, 512, 128), info)
    xh_a = _decode(tau, acts_a, W_dec, bd)
    xh_b = _decode(tau, acts_b, W_dec, bd)
    return jnp.concatenate([xh_a, xh_b], axis=0)
